# Initial kernel scaffold; baseline (speedup 1.0000x reference)
import jax
import jax.numpy as jnp
from jax.experimental import pallas as pl


def kernel(outputs, targets, weight):
    raise NotImplementedError("write your pallas kernel here")


# speedup vs baseline: 1.2931x; 1.2931x over previous
import functools

import jax
import jax.numpy as jnp
from jax.experimental import pallas as pl
from jax.experimental.pallas import tpu as pltpu

_IGNORE_INDEX = -100

_BLOCK_N = 2048
_CHUNK_N = 256
_BLOCK_V = 640


def _ce_kernel(nv_tiles, x_ref, t_ref, w_ref, loss_ref, cnt_ref,
               m_ref, s_ref, tgt_ref):
    j = pl.program_id(1)

    @pl.when(j == 0)
    def _init():
        m_ref[...] = jnp.full(m_ref.shape, -jnp.inf, dtype=jnp.float32)
        s_ref[...] = jnp.zeros(s_ref.shape, dtype=jnp.float32)
        tgt_ref[...] = jnp.zeros(tgt_ref.shape, dtype=jnp.float32)

    col0 = j * _BLOCK_V
    iota_v = jax.lax.broadcasted_iota(jnp.int32, (_BLOCK_V, _CHUNK_N), 0)

    for r in range(_BLOCK_N // _CHUNK_N):
        sl = slice(r * _CHUNK_N, (r + 1) * _CHUNK_N)
        xr = x_ref[:, sl]
        lt = jax.lax.dot_general(
            w_ref[...], xr,
            dimension_numbers=(((1,), (0,)), ((), ())),
            preferred_element_type=jnp.float32)
        t_row = t_ref[0, :, sl]

        m_old = m_ref[:, sl]
        lm = jnp.max(lt, axis=0, keepdims=True)
        m_new = jnp.maximum(m_old, lm)
        p = jnp.exp(lt - m_new)
        s_new = s_ref[:, sl] * jnp.exp(m_old - m_new) + jnp.sum(
            p, axis=0, keepdims=True)
        hit = (iota_v + col0) == t_row
        tgt_new = tgt_ref[:, sl] + jnp.sum(
            jnp.where(hit, lt, 0.0), axis=0, keepdims=True)

        m_ref[:, sl] = m_new
        s_ref[:, sl] = s_new
        tgt_ref[:, sl] = tgt_new

    @pl.when(j == nv_tiles - 1)
    def _finalize():
        t_all = t_ref[0, :, :]
        valid = t_all != _IGNORE_INDEX
        lse = m_ref[...] + jnp.log(s_ref[...])
        loss = jnp.where(valid, lse - tgt_ref[...], 0.0)
        cnt = jnp.where(valid, 1.0, 0.0)
        l_acc = loss[:, 0:128]
        c_acc = cnt[:, 0:128]
        for k in range(1, _BLOCK_N // 128):
            ksl = slice(k * 128, (k + 1) * 128)
            l_acc = l_acc + loss[:, ksl]
            c_acc = c_acc + cnt[:, ksl]
        loss_ref[...] = l_acc[None]
        cnt_ref[...] = c_acc[None]


@jax.jit
def kernel(outputs, targets, weight):
    B, S, D = outputs.shape
    V = weight.shape[0]
    N = B * S
    nb = N // _BLOCK_N
    nv = V // _BLOCK_V

    x_t = outputs.reshape(N, D).T.astype(jnp.bfloat16)
    w = weight.astype(jnp.bfloat16)
    t = targets.reshape(nb, 1, _BLOCK_N)

    grid = (nb, nv)
    loss_parts, cnt_parts = pl.pallas_call(
        functools.partial(_ce_kernel, nv),
        grid=grid,
        in_specs=[
            pl.BlockSpec((D, _BLOCK_N), lambda i, j: (0, i)),
            pl.BlockSpec((1, 1, _BLOCK_N), lambda i, j: (i, 0, 0)),
            pl.BlockSpec((_BLOCK_V, D), lambda i, j: (j, 0)),
        ],
        out_specs=[
            pl.BlockSpec((1, 1, 128), lambda i, j: (i, 0, 0)),
            pl.BlockSpec((1, 1, 128), lambda i, j: (i, 0, 0)),
        ],
        out_shape=[
            jax.ShapeDtypeStruct((nb, 1, 128), jnp.float32),
            jax.ShapeDtypeStruct((nb, 1, 128), jnp.float32),
        ],
        scratch_shapes=[
            pltpu.VMEM((1, _BLOCK_N), jnp.float32),
            pltpu.VMEM((1, _BLOCK_N), jnp.float32),
            pltpu.VMEM((1, _BLOCK_N), jnp.float32),
        ],
        compiler_params=pltpu.CompilerParams(
            dimension_semantics=("parallel", "arbitrary"),
        ),
    )(x_t, t, w)

    total = jnp.sum(loss_parts)
    cnt = jnp.sum(cnt_parts)
    return total / jnp.maximum(cnt, 1.0)


# speedup vs baseline: 1.5167x; 1.1729x over previous
import functools

import jax
import jax.numpy as jnp
from jax.experimental import pallas as pl
from jax.experimental.pallas import tpu as pltpu

_IGNORE_INDEX = -100

_BLOCK_N = 2048
_CHUNK_N = 256
_BLOCK_V = 640


def _ce_kernel(nv_tiles, x_ref, t_ref, w_ref, loss_ref, cnt_ref,
               m_ref, s_ref, tgt_ref):
    j = pl.program_id(1)

    @pl.when(j == 0)
    def _init():
        m_ref[...] = jnp.full(m_ref.shape, -jnp.inf, dtype=jnp.float32)
        s_ref[...] = jnp.zeros(s_ref.shape, dtype=jnp.float32)
        tgt_ref[...] = jnp.zeros(tgt_ref.shape, dtype=jnp.float32)

    col0 = j * _BLOCK_V
    iota_v = jax.lax.broadcasted_iota(jnp.int32, (_BLOCK_V, _CHUNK_N), 0)
    wb = w_ref[...].astype(jnp.bfloat16)

    for r in range(_BLOCK_N // _CHUNK_N):
        sl = slice(r * _CHUNK_N, (r + 1) * _CHUNK_N)
        xr = x_ref[:, sl]
        lt = jax.lax.dot_general(
            wb, xr,
            dimension_numbers=(((1,), (0,)), ((), ())),
            preferred_element_type=jnp.float32)
        t_row = t_ref[0, :, sl]

        m_old = m_ref[:, sl]
        lm = jnp.max(lt, axis=0, keepdims=True)
        m_new = jnp.maximum(m_old, lm)
        p = jnp.exp(lt - m_new)
        s_new = s_ref[:, sl] * jnp.exp(m_old - m_new) + jnp.sum(
            p, axis=0, keepdims=True)
        hit = (iota_v + col0) == t_row
        tgt_new = tgt_ref[:, sl] + jnp.sum(
            jnp.where(hit, lt, 0.0), axis=0, keepdims=True)

        m_ref[:, sl] = m_new
        s_ref[:, sl] = s_new
        tgt_ref[:, sl] = tgt_new

    @pl.when(j == nv_tiles - 1)
    def _finalize():
        t_all = t_ref[0, :, :]
        valid = t_all != _IGNORE_INDEX
        lse = m_ref[...] + jnp.log(s_ref[...])
        loss = jnp.where(valid, lse - tgt_ref[...], 0.0)
        cnt = jnp.where(valid, 1.0, 0.0)
        l_acc = loss[:, 0:128]
        c_acc = cnt[:, 0:128]
        for k in range(1, _BLOCK_N // 128):
            ksl = slice(k * 128, (k + 1) * 128)
            l_acc = l_acc + loss[:, ksl]
            c_acc = c_acc + cnt[:, ksl]
        loss_ref[...] = l_acc[None]
        cnt_ref[...] = c_acc[None]


@jax.jit
def kernel(outputs, targets, weight):
    B, S, D = outputs.shape
    V = weight.shape[0]
    N = B * S
    nb = N // _BLOCK_N
    nv = V // _BLOCK_V

    x_t = outputs.reshape(N, D).T.astype(jnp.bfloat16)
    t = targets.reshape(nb, 1, _BLOCK_N)

    grid = (nb, nv)
    loss_parts, cnt_parts = pl.pallas_call(
        functools.partial(_ce_kernel, nv),
        grid=grid,
        in_specs=[
            pl.BlockSpec((D, _BLOCK_N), lambda i, j: (0, i)),
            pl.BlockSpec((1, 1, _BLOCK_N), lambda i, j: (i, 0, 0)),
            pl.BlockSpec((_BLOCK_V, D), lambda i, j: (j, 0)),
        ],
        out_specs=[
            pl.BlockSpec((1, 1, 128), lambda i, j: (i, 0, 0)),
            pl.BlockSpec((1, 1, 128), lambda i, j: (i, 0, 0)),
        ],
        out_shape=[
            jax.ShapeDtypeStruct((nb, 1, 128), jnp.float32),
            jax.ShapeDtypeStruct((nb, 1, 128), jnp.float32),
        ],
        scratch_shapes=[
            pltpu.VMEM((1, _BLOCK_N), jnp.float32),
            pltpu.VMEM((1, _BLOCK_N), jnp.float32),
            pltpu.VMEM((1, _BLOCK_N), jnp.float32),
        ],
        compiler_params=pltpu.CompilerParams(
            dimension_semantics=("parallel", "arbitrary"),
        ),
    )(x_t, t, weight)

    total = jnp.sum(loss_parts)
    cnt = jnp.sum(cnt_parts)
    return total / jnp.maximum(cnt, 1.0)


# speedup vs baseline: 1.5426x; 1.0171x over previous
import functools

import jax
import jax.numpy as jnp
from jax.experimental import pallas as pl
from jax.experimental.pallas import tpu as pltpu

_IGNORE_INDEX = -100

_CHUNK_N = 256
_BLOCK_V = 640


def _ce_kernel(n_tok, nv_half, x_ref, t_ref, w_ref, m_out, s_out, tgt_out,
               m_ref, s_ref, tgt_ref):
    j = pl.program_id(1)
    col0 = (pl.program_id(0) * nv_half + j) * _BLOCK_V

    @pl.when(j == 0)
    def _init():
        m_ref[...] = jnp.full(m_ref.shape, -jnp.inf, dtype=jnp.float32)
        s_ref[...] = jnp.zeros(s_ref.shape, dtype=jnp.float32)
        tgt_ref[...] = jnp.zeros(tgt_ref.shape, dtype=jnp.float32)

    iota_v = jax.lax.broadcasted_iota(jnp.int32, (_BLOCK_V, _CHUNK_N), 0)
    wb = w_ref[...].astype(jnp.bfloat16)

    for r in range(n_tok // _CHUNK_N):
        sl = slice(r * _CHUNK_N, (r + 1) * _CHUNK_N)
        xr = x_ref[:, sl]
        lt = jax.lax.dot_general(
            wb, xr,
            dimension_numbers=(((1,), (0,)), ((), ())),
            preferred_element_type=jnp.float32)
        t_row = t_ref[0, :, sl]

        m_old = m_ref[:, sl]
        lm = jnp.max(lt, axis=0, keepdims=True)
        m_new = jnp.maximum(m_old, lm)
        p = jnp.exp(lt - m_new)
        s_new = s_ref[:, sl] * jnp.exp(m_old - m_new) + jnp.sum(
            p, axis=0, keepdims=True)
        hit = (iota_v + col0) == t_row
        tgt_new = tgt_ref[:, sl] + jnp.sum(
            jnp.where(hit, lt, 0.0), axis=0, keepdims=True)

        m_ref[:, sl] = m_new
        s_ref[:, sl] = s_new
        tgt_ref[:, sl] = tgt_new

    @pl.when(j == nv_half - 1)
    def _finalize():
        m_out[...] = m_ref[...][None]
        s_out[...] = s_ref[...][None]
        tgt_out[...] = tgt_ref[...][None]


def _merge_kernel(n_tok, m_ref, s_ref, tgt_ref, t_ref, loss_out, cnt_out):
    m0, m1 = m_ref[0], m_ref[1]
    mm = jnp.maximum(m0, m1)
    s = s_ref[0] * jnp.exp(m0 - mm) + s_ref[1] * jnp.exp(m1 - mm)
    lse = mm + jnp.log(s)
    tgt = tgt_ref[0] + tgt_ref[1]
    valid = t_ref[0] != _IGNORE_INDEX
    loss = jnp.where(valid, lse - tgt, 0.0)
    cnt = jnp.where(valid, 1.0, 0.0)
    l_acc = loss[:, 0:128]
    c_acc = cnt[:, 0:128]
    for k in range(1, n_tok // 128):
        ksl = slice(k * 128, (k + 1) * 128)
        l_acc = l_acc + loss[:, ksl]
        c_acc = c_acc + cnt[:, ksl]
    loss_out[...] = l_acc
    cnt_out[...] = c_acc


@jax.jit
def kernel(outputs, targets, weight):
    B, S, D = outputs.shape
    V = weight.shape[0]
    N = B * S
    nv_half = V // _BLOCK_V // 2

    x_t = outputs.reshape(N, D).T.astype(jnp.bfloat16)
    t = targets.reshape(1, 1, N)

    grid = (2, nv_half)
    stat_sds = jax.ShapeDtypeStruct((2, 1, N), jnp.float32)
    m_p, s_p, tgt_p = pl.pallas_call(
        functools.partial(_ce_kernel, N, nv_half),
        grid=grid,
        in_specs=[
            pl.BlockSpec((D, N), lambda i, j: (0, 0)),
            pl.BlockSpec((1, 1, N), lambda i, j: (0, 0, 0)),
            pl.BlockSpec((_BLOCK_V, D), lambda i, j: (i * nv_half + j, 0)),
        ],
        out_specs=[
            pl.BlockSpec((1, 1, N), lambda i, j: (i, 0, 0)),
            pl.BlockSpec((1, 1, N), lambda i, j: (i, 0, 0)),
            pl.BlockSpec((1, 1, N), lambda i, j: (i, 0, 0)),
        ],
        out_shape=[stat_sds, stat_sds, stat_sds],
        scratch_shapes=[
            pltpu.VMEM((1, N), jnp.float32),
            pltpu.VMEM((1, N), jnp.float32),
            pltpu.VMEM((1, N), jnp.float32),
        ],
        compiler_params=pltpu.CompilerParams(
            dimension_semantics=("parallel", "arbitrary"),
            vmem_limit_bytes=56 * 1024 * 1024,
        ),
    )(x_t, t, weight)

    loss_parts, cnt_parts = pl.pallas_call(
        functools.partial(_merge_kernel, N),
        out_shape=[
            jax.ShapeDtypeStruct((1, 128), jnp.float32),
            jax.ShapeDtypeStruct((1, 128), jnp.float32),
        ],
    )(m_p, s_p, tgt_p, t)

    total = jnp.sum(loss_parts)
    cnt = jnp.sum(cnt_parts)
    return total / jnp.maximum(cnt, 1.0)
